# baseline (speedup 1.0000x reference)
import jax
import jax.numpy as jnp
from jax import lax
from jax.experimental import pallas as pl
from jax.experimental.pallas import tpu as pltpu
from jax.experimental.pallas import tpu_sc as plsc

N_NODES = 10000
D = 128

NUM_CORES = 2
NUM_SUBCORES = 16

CHUNK = 128
CHUNKS_PER_TEC = 160
EDGES_PADDED = NUM_SUBCORES * CHUNKS_PER_TEC * CHUNK
PHASE = 80
NPHASE = CHUNKS_PER_TEC // PHASE
AGG_ROWS = 10240
ROWS_PER_TEC = AGG_ROWS // NUM_SUBCORES
DUMMY_NODE = N_NODES


def _sc_body(x_hbm, row_hbm, col_hbm, zer_hbm, ones_hbm,
             agg_out, deg_out,
             row_v, col_v, gbuf_a, acc_sp,
             sem_a):
    cid = lax.axis_index("c")
    sid = lax.axis_index("s")

    r0 = sid * ROWS_PER_TEC
    pltpu.sync_copy(zer_hbm, acc_sp.at[pl.ds(r0, ROWS_PER_TEC)])
    plsc.subcore_barrier()

    @pl.when(cid == 0)
    def _agg():
        for p in range(NPHASE):
            pltpu.sync_copy(row_hbm.at[sid, pl.ds(p * PHASE, PHASE)], row_v)
            pltpu.sync_copy(col_hbm.at[sid, pl.ds(p * PHASE, PHASE)], col_v)

            def body(i, _):
                pltpu.make_async_copy(x_hbm.at[row_v.at[i]], gbuf_a, sem_a).start()
                pltpu.make_async_copy(x_hbm.at[row_v.at[i]], gbuf_a, sem_a).wait()
                pltpu.sync_copy(gbuf_a, acc_sp.at[col_v.at[i]], add=True)
                return 0

            lax.fori_loop(0, PHASE, body, 0)

    @pl.when(cid == 1)
    def _deg():
        pltpu.sync_copy(ones_hbm, gbuf_a)
        for p in range(NPHASE):
            pltpu.sync_copy(col_hbm.at[sid, pl.ds(p * PHASE, PHASE)], col_v)

            def body(i, _):
                pltpu.sync_copy(gbuf_a, acc_sp.at[col_v.at[i]], add=True)
                return 0

            lax.fori_loop(0, PHASE, body, 0)

    plsc.subcore_barrier()

    @pl.when(cid == 0)
    def _out_agg():
        pltpu.sync_copy(acc_sp.at[pl.ds(r0, ROWS_PER_TEC)],
                        agg_out.at[pl.ds(r0, ROWS_PER_TEC)])

    @pl.when(cid == 1)
    def _out_deg():
        pltpu.sync_copy(acc_sp.at[pl.ds(r0, ROWS_PER_TEC)],
                        deg_out.at[pl.ds(r0, ROWS_PER_TEC)])


@jax.jit
def _sc_aggregate(x, row3d, col3d, zer, ones):
    mesh = plsc.VectorSubcoreMesh(core_axis_name="c", subcore_axis_name="s")
    return pl.kernel(
        _sc_body,
        out_type=(
            jax.ShapeDtypeStruct((AGG_ROWS, D), jnp.float32),
            jax.ShapeDtypeStruct((AGG_ROWS, D), jnp.float32),
        ),
        mesh=mesh,
        scratch_types=[
            pltpu.VMEM((PHASE, CHUNK), jnp.int32),
            pltpu.VMEM((PHASE, CHUNK), jnp.int32),
            pltpu.VMEM((CHUNK, D), jnp.float32),
            pltpu.VMEM_SHARED((AGG_ROWS, D), jnp.float32),
            pltpu.SemaphoreType.DMA,
        ],
    )(x, row3d, col3d, zer, ones)


ROW_TILE = 400


def _tc_dense_body(x_ref, agg_ref, deg_ref, ws_ref, wd_ref, wf_ref,
                   bs_ref, bd_ref, bf_ref, out_ref):
    xb = x_ref[...]
    aggb = agg_ref[...]
    degb = deg_ref[:, 0:1]
    dn = (((1,), (1,)), ((), ()))
    t_self = lax.dot_general(xb, wf_ref[...], dn, preferred_element_type=jnp.float32)
    t_dst = lax.dot_general(xb, wd_ref[...], dn, preferred_element_type=jnp.float32)
    t_src = lax.dot_general(aggb, ws_ref[...], dn, preferred_element_type=jnp.float32)
    out_ref[...] = (t_self + t_src - degb * t_dst
                    + bf_ref[...] + degb * (bs_ref[...] - bd_ref[...]))


@jax.jit
def _tc_dense(x, agg, deg, W_src, W_dst, W_self, b_src, b_dst, b_self):
    grid = (N_NODES // ROW_TILE,)
    return pl.pallas_call(
        _tc_dense_body,
        grid=grid,
        in_specs=[
            pl.BlockSpec((ROW_TILE, D), lambda i: (i, 0)),
            pl.BlockSpec((ROW_TILE, D), lambda i: (i, 0)),
            pl.BlockSpec((ROW_TILE, D), lambda i: (i, 0)),
            pl.BlockSpec((D, D), lambda i: (0, 0)),
            pl.BlockSpec((D, D), lambda i: (0, 0)),
            pl.BlockSpec((D, D), lambda i: (0, 0)),
            pl.BlockSpec((1, D), lambda i: (0, 0)),
            pl.BlockSpec((1, D), lambda i: (0, 0)),
            pl.BlockSpec((1, D), lambda i: (0, 0)),
        ],
        out_specs=pl.BlockSpec((ROW_TILE, D), lambda i: (i, 0)),
        out_shape=jax.ShapeDtypeStruct((N_NODES, D), jnp.float32),
    )(x, agg, deg, W_src, W_dst, W_self, b_src, b_dst, b_self)


def kernel(x, edge_index, W_src, b_src, W_dst, b_dst, W_self, b_self):
    row = edge_index[0].astype(jnp.int32)
    col = edge_index[1].astype(jnp.int32)
    pad = EDGES_PADDED - row.shape[0]
    row_p = jnp.concatenate([row, jnp.zeros((pad,), jnp.int32)])
    col_p = jnp.concatenate([col, jnp.full((pad,), DUMMY_NODE, jnp.int32)])
    row3d = row_p.reshape(NUM_SUBCORES, CHUNKS_PER_TEC, CHUNK)
    col3d = col_p.reshape(NUM_SUBCORES, CHUNKS_PER_TEC, CHUNK)

    zer = jnp.zeros((ROWS_PER_TEC, D), jnp.float32)
    ones = jnp.ones((CHUNK, D), jnp.float32)

    agg, deg = _sc_aggregate(x, row3d, col3d, zer, ones)
    return _tc_dense(x, agg, deg, W_src, W_dst, W_self,
                     b_src.reshape(1, D), b_dst.reshape(1, D),
                     b_self.reshape(1, D))


# speedup vs baseline: 1.2626x; 1.2626x over previous
import jax
import jax.numpy as jnp
from jax import lax
from jax.experimental import pallas as pl
from jax.experimental.pallas import tpu as pltpu
from jax.experimental.pallas import tpu_sc as plsc

N_NODES = 10000
D = 128

NUM_CORES = 2
NUM_SUBCORES = 16

CHUNK = 128
CHUNKS_PER_TEC = 160
EDGES_PADDED = NUM_SUBCORES * CHUNKS_PER_TEC * CHUNK
PHASE = 40
NPHASE = CHUNKS_PER_TEC // PHASE
AGG_ROWS = 10240
ROWS_PER_TEC = AGG_ROWS // NUM_SUBCORES
DUMMY_NODE = N_NODES


def _sc_body(x_hbm, row_hbm, col_hbm, zer_hbm, ones_hbm,
             agg_out, deg_out,
             row_v, col_v, gbuf_a, gbuf_b, acc_sp,
             sem_a, sem_b):
    cid = lax.axis_index("c")
    sid = lax.axis_index("s")

    r0 = sid * ROWS_PER_TEC
    pltpu.sync_copy(zer_hbm, acc_sp.at[pl.ds(r0, ROWS_PER_TEC)])
    plsc.subcore_barrier()

    @pl.when(cid == 1)
    def _agg():
        for p in range(NPHASE):
            pltpu.sync_copy(row_hbm.at[sid, pl.ds(p * PHASE, PHASE)], row_v)
            pltpu.sync_copy(col_hbm.at[sid, pl.ds(p * PHASE, PHASE)], col_v)
            pltpu.async_copy(x_hbm.at[row_v.at[0]], gbuf_a, sem_a)

            def body(k, _):
                i0 = 2 * k
                i1 = 2 * k + 1
                pltpu.async_copy(x_hbm.at[row_v.at[i1]], gbuf_b, sem_b)
                pltpu.make_async_copy(x_hbm.at[row_v.at[i0]], gbuf_a, sem_a).wait()
                pltpu.sync_copy(gbuf_a, acc_sp.at[col_v.at[i0]], add=True)
                nxt = jnp.minimum(i0 + 2, PHASE - 1)
                pltpu.async_copy(x_hbm.at[row_v.at[nxt]], gbuf_a, sem_a)
                pltpu.make_async_copy(x_hbm.at[row_v.at[i1]], gbuf_b, sem_b).wait()
                pltpu.sync_copy(gbuf_b, acc_sp.at[col_v.at[i1]], add=True)
                return 0

            lax.fori_loop(0, PHASE // 2, body, 0)
            pltpu.make_async_copy(x_hbm.at[row_v.at[PHASE - 1]], gbuf_a, sem_a).wait()

    @pl.when(cid == 0)
    def _deg():
        pltpu.sync_copy(ones_hbm, gbuf_a)
        for p in range(NPHASE):
            pltpu.sync_copy(col_hbm.at[sid, pl.ds(p * PHASE, PHASE)], col_v)

            def body(i, _):
                pltpu.sync_copy(gbuf_a, acc_sp.at[col_v.at[i]], add=True)
                return 0

            lax.fori_loop(0, PHASE, body, 0)

    plsc.subcore_barrier()

    @pl.when(cid == 1)
    def _out_agg():
        pltpu.sync_copy(acc_sp.at[pl.ds(r0, ROWS_PER_TEC)],
                        agg_out.at[pl.ds(r0, ROWS_PER_TEC)])

    @pl.when(cid == 0)
    def _out_deg():
        pltpu.sync_copy(acc_sp.at[pl.ds(r0, ROWS_PER_TEC)],
                        deg_out.at[pl.ds(r0, ROWS_PER_TEC)])


@jax.jit
def _sc_aggregate(x, row3d, col3d, zer, ones):
    mesh = plsc.VectorSubcoreMesh(core_axis_name="c", subcore_axis_name="s")
    return pl.kernel(
        _sc_body,
        out_type=(
            jax.ShapeDtypeStruct((AGG_ROWS, D), jnp.float32),
            jax.ShapeDtypeStruct((AGG_ROWS, D), jnp.float32),
        ),
        mesh=mesh,
        scratch_types=[
            pltpu.VMEM((PHASE, CHUNK), jnp.int32),
            pltpu.VMEM((PHASE, CHUNK), jnp.int32),
            pltpu.VMEM((CHUNK, D), jnp.float32),
            pltpu.VMEM((CHUNK, D), jnp.float32),
            pltpu.VMEM_SHARED((AGG_ROWS, D), jnp.float32),
            pltpu.SemaphoreType.DMA,
            pltpu.SemaphoreType.DMA,
        ],
    )(x, row3d, col3d, zer, ones)


ROW_TILE = 400


def _tc_dense_body(x_ref, agg_ref, deg_ref, ws_ref, wd_ref, wf_ref,
                   bs_ref, bd_ref, bf_ref, out_ref):
    xb = x_ref[...]
    aggb = agg_ref[...]
    degb = deg_ref[:, 0:1]
    dn = (((1,), (1,)), ((), ()))
    t_self = lax.dot_general(xb, wf_ref[...], dn, preferred_element_type=jnp.float32)
    t_dst = lax.dot_general(xb, wd_ref[...], dn, preferred_element_type=jnp.float32)
    t_src = lax.dot_general(aggb, ws_ref[...], dn, preferred_element_type=jnp.float32)
    out_ref[...] = (t_self + t_src - degb * t_dst
                    + bf_ref[...] + degb * (bs_ref[...] - bd_ref[...]))


@jax.jit
def _tc_dense(x, agg, deg, W_src, W_dst, W_self, b_src, b_dst, b_self):
    grid = (N_NODES // ROW_TILE,)
    return pl.pallas_call(
        _tc_dense_body,
        grid=grid,
        in_specs=[
            pl.BlockSpec((ROW_TILE, D), lambda i: (i, 0)),
            pl.BlockSpec((ROW_TILE, D), lambda i: (i, 0)),
            pl.BlockSpec((ROW_TILE, D), lambda i: (i, 0)),
            pl.BlockSpec((D, D), lambda i: (0, 0)),
            pl.BlockSpec((D, D), lambda i: (0, 0)),
            pl.BlockSpec((D, D), lambda i: (0, 0)),
            pl.BlockSpec((1, D), lambda i: (0, 0)),
            pl.BlockSpec((1, D), lambda i: (0, 0)),
            pl.BlockSpec((1, D), lambda i: (0, 0)),
        ],
        out_specs=pl.BlockSpec((ROW_TILE, D), lambda i: (i, 0)),
        out_shape=jax.ShapeDtypeStruct((N_NODES, D), jnp.float32),
    )(x, agg, deg, W_src, W_dst, W_self, b_src, b_dst, b_self)


def kernel(x, edge_index, W_src, b_src, W_dst, b_dst, W_self, b_self):
    row = edge_index[0].astype(jnp.int32)
    col = edge_index[1].astype(jnp.int32)
    pad = EDGES_PADDED - row.shape[0]
    row_p = jnp.concatenate([row, jnp.zeros((pad,), jnp.int32)])
    col_p = jnp.concatenate([col, jnp.full((pad,), DUMMY_NODE, jnp.int32)])
    row3d = row_p.reshape(NUM_SUBCORES, CHUNKS_PER_TEC, CHUNK)
    col3d = col_p.reshape(NUM_SUBCORES, CHUNKS_PER_TEC, CHUNK)

    zer = jnp.zeros((ROWS_PER_TEC, D), jnp.float32)
    ones = jnp.ones((CHUNK, D), jnp.float32)

    agg, deg = _sc_aggregate(x, row3d, col3d, zer, ones)
    return _tc_dense(x, agg, deg, W_src, W_dst, W_self,
                     b_src.reshape(1, D), b_dst.reshape(1, D),
                     b_self.reshape(1, D))


# speedup vs baseline: 1.2669x; 1.0034x over previous
import jax
import jax.numpy as jnp
from jax import lax
from jax.experimental import pallas as pl
from jax.experimental.pallas import tpu as pltpu
from jax.experimental.pallas import tpu_sc as plsc

N_NODES = 10000
D = 128

NUM_CORES = 2
NUM_SUBCORES = 16

CHUNK = 128
CHUNKS_PER_TEC = 160
EDGES_PADDED = NUM_SUBCORES * CHUNKS_PER_TEC * CHUNK
PHASES = (56, 56, 48)
PHASE_BUF = max(PHASES)
AGG_ROWS = 10240
ROWS_PER_TEC = AGG_ROWS // NUM_SUBCORES
DUMMY_NODE = N_NODES


def _sc_body(x_hbm, row_hbm, col_hbm, zer_hbm, ones_hbm,
             agg_out, deg_out,
             row_v, col_v, gbuf_a, gbuf_b, acc_sp,
             sem_a, sem_b):
    cid = lax.axis_index("c")
    sid = lax.axis_index("s")

    r0 = sid * ROWS_PER_TEC
    pltpu.sync_copy(zer_hbm, acc_sp.at[pl.ds(r0, ROWS_PER_TEC)])
    plsc.subcore_barrier()

    @pl.when(cid == 1)
    def _agg():
        off = 0
        for ln in PHASES:
            pltpu.sync_copy(row_hbm.at[sid, pl.ds(off, ln)], row_v.at[pl.ds(0, ln)])
            pltpu.sync_copy(col_hbm.at[sid, pl.ds(off, ln)], col_v.at[pl.ds(0, ln)])
            pltpu.async_copy(x_hbm.at[row_v.at[0]], gbuf_a, sem_a)

            def body(k, _):
                i0 = 2 * k
                i1 = 2 * k + 1
                pltpu.async_copy(x_hbm.at[row_v.at[i1]], gbuf_b, sem_b)
                pltpu.make_async_copy(x_hbm.at[row_v.at[i0]], gbuf_a, sem_a).wait()
                pltpu.sync_copy(gbuf_a, acc_sp.at[col_v.at[i0]], add=True)
                nxt = jnp.minimum(i0 + 2, ln - 1)
                pltpu.async_copy(x_hbm.at[row_v.at[nxt]], gbuf_a, sem_a)
                pltpu.make_async_copy(x_hbm.at[row_v.at[i1]], gbuf_b, sem_b).wait()
                pltpu.sync_copy(gbuf_b, acc_sp.at[col_v.at[i1]], add=True)
                return 0

            lax.fori_loop(0, ln // 2, body, 0)
            pltpu.make_async_copy(x_hbm.at[row_v.at[ln - 1]], gbuf_a, sem_a).wait()
            off += ln

    @pl.when(cid == 0)
    def _deg():
        pltpu.sync_copy(ones_hbm, gbuf_a)
        off = 0
        for ln in PHASES:
            pltpu.sync_copy(col_hbm.at[sid, pl.ds(off, ln)], col_v.at[pl.ds(0, ln)])

            def body(i, _):
                pltpu.sync_copy(gbuf_a, acc_sp.at[col_v.at[i]], add=True)
                return 0

            lax.fori_loop(0, ln, body, 0)
            off += ln

    plsc.subcore_barrier()

    @pl.when(cid == 1)
    def _out_agg():
        pltpu.sync_copy(acc_sp.at[pl.ds(r0, ROWS_PER_TEC)],
                        agg_out.at[pl.ds(r0, ROWS_PER_TEC)])

    @pl.when(cid == 0)
    def _out_deg():
        pltpu.sync_copy(acc_sp.at[pl.ds(r0, ROWS_PER_TEC)],
                        deg_out.at[pl.ds(r0, ROWS_PER_TEC)])


@jax.jit
def _sc_aggregate(x, row3d, col3d, zer, ones):
    mesh = plsc.VectorSubcoreMesh(core_axis_name="c", subcore_axis_name="s")
    return pl.kernel(
        _sc_body,
        out_type=(
            jax.ShapeDtypeStruct((AGG_ROWS, D), jnp.float32),
            jax.ShapeDtypeStruct((AGG_ROWS, D), jnp.float32),
        ),
        mesh=mesh,
        scratch_types=[
            pltpu.VMEM((PHASE_BUF, CHUNK), jnp.int32),
            pltpu.VMEM((PHASE_BUF, CHUNK), jnp.int32),
            pltpu.VMEM((CHUNK, D), jnp.float32),
            pltpu.VMEM((CHUNK, D), jnp.float32),
            pltpu.VMEM_SHARED((AGG_ROWS, D), jnp.float32),
            pltpu.SemaphoreType.DMA,
            pltpu.SemaphoreType.DMA,
        ],
    )(x, row3d, col3d, zer, ones)


ROW_TILE = 400


def _tc_dense_body(x_ref, agg_ref, deg_ref, ws_ref, wd_ref, wf_ref,
                   bs_ref, bd_ref, bf_ref, out_ref):
    xb = x_ref[...]
    aggb = agg_ref[...]
    degb = deg_ref[:, 0:1]
    dn = (((1,), (1,)), ((), ()))
    t_self = lax.dot_general(xb, wf_ref[...], dn, preferred_element_type=jnp.float32)
    t_dst = lax.dot_general(xb, wd_ref[...], dn, preferred_element_type=jnp.float32)
    t_src = lax.dot_general(aggb, ws_ref[...], dn, preferred_element_type=jnp.float32)
    out_ref[...] = (t_self + t_src - degb * t_dst
                    + bf_ref[...] + degb * (bs_ref[...] - bd_ref[...]))


@jax.jit
def _tc_dense(x, agg, deg, W_src, W_dst, W_self, b_src, b_dst, b_self):
    grid = (N_NODES // ROW_TILE,)
    return pl.pallas_call(
        _tc_dense_body,
        grid=grid,
        in_specs=[
            pl.BlockSpec((ROW_TILE, D), lambda i: (i, 0)),
            pl.BlockSpec((ROW_TILE, D), lambda i: (i, 0)),
            pl.BlockSpec((ROW_TILE, D), lambda i: (i, 0)),
            pl.BlockSpec((D, D), lambda i: (0, 0)),
            pl.BlockSpec((D, D), lambda i: (0, 0)),
            pl.BlockSpec((D, D), lambda i: (0, 0)),
            pl.BlockSpec((1, D), lambda i: (0, 0)),
            pl.BlockSpec((1, D), lambda i: (0, 0)),
            pl.BlockSpec((1, D), lambda i: (0, 0)),
        ],
        out_specs=pl.BlockSpec((ROW_TILE, D), lambda i: (i, 0)),
        out_shape=jax.ShapeDtypeStruct((N_NODES, D), jnp.float32),
    )(x, agg, deg, W_src, W_dst, W_self, b_src, b_dst, b_self)


def kernel(x, edge_index, W_src, b_src, W_dst, b_dst, W_self, b_self):
    row = edge_index[0].astype(jnp.int32)
    col = edge_index[1].astype(jnp.int32)
    pad = EDGES_PADDED - row.shape[0]
    row_p = jnp.concatenate([row, jnp.zeros((pad,), jnp.int32)])
    col_p = jnp.concatenate([col, jnp.full((pad,), DUMMY_NODE, jnp.int32)])
    row3d = row_p.reshape(NUM_SUBCORES, CHUNKS_PER_TEC, CHUNK)
    col3d = col_p.reshape(NUM_SUBCORES, CHUNKS_PER_TEC, CHUNK)

    zer = jnp.zeros((ROWS_PER_TEC, D), jnp.float32)
    ones = jnp.ones((CHUNK, D), jnp.float32)

    agg, deg = _sc_aggregate(x, row3d, col3d, zer, ones)
    return _tc_dense(x, agg, deg, W_src, W_dst, W_self,
                     b_src.reshape(1, D), b_dst.reshape(1, D),
                     b_self.reshape(1, D))
